# baseline (speedup 1.0000x reference)
import functools

import jax
import jax.numpy as jnp
from jax import lax
from jax.experimental import pallas as pl
from jax.experimental.pallas import tpu as pltpu
from jax.experimental.pallas import tpu_sc as plsc

B = 16384
D = 256
TOK = 4
CD = 64
NC = 2
NS = 16
NW = NC * NS
ROWS_PER_W = B // NW
CHUNK = 128
NCHUNKS = ROWS_PER_W // CHUNK
L = 16


def _rsqrt_vec(x):
    i = lax.bitcast_convert_type(x, jnp.int32)
    i = jnp.int32(0x5F3759DF) - lax.shift_right_logical(i, 1)
    y = lax.bitcast_convert_type(i, jnp.float32)
    for _ in range(3):
        y = y * (jnp.float32(1.5) - jnp.float32(0.5) * x * y * y)
    return y


_GATHER_DN = lax.GatherDimensionNumbers(
    offset_dims=(), collapsed_slice_dims=(0,), start_index_map=(0,))


def _perm(x, idx):
    return lax.gather(x, idx[:, None], _GATHER_DN, slice_sizes=(1,),
                      mode=lax.GatherScatterMode.PROMISE_IN_BOUNDS)


def _lane_sum(x, bfly):
    for idx in bfly:
        x = x + _perm(x, idx)
    return x


def _sc_body(ids_hbm, w_hbm, b_hbm, embed_hbm, out_hbm, idx_v, buf_v, wb_v, sem):
    wid = lax.axis_index("s") * NC + lax.axis_index("c")
    base = wid * ROWS_PER_W

    pltpu.sync_copy(w_hbm, wb_v.at[pl.ds(0, CD)])
    pltpu.sync_copy(b_hbm, wb_v.at[pl.ds(CD, CD)])
    w_regs = [wb_v[pl.ds(L * k, L)] for k in range(4)]
    b_regs = [wb_v[pl.ds(CD + L * k, L)] for k in range(4)]

    inv_cd = jnp.float32(1.0 / CD)
    lane = lax.iota(jnp.int32, L)
    bfly = [lax.bitwise_xor(lane, jnp.int32(sh)) for sh in (8, 4, 2, 1)]

    def row_body(r, carry):
        for t in range(TOK):
            u = [buf_v[r, pl.ds(t * CD + L * k, L)] for k in range(4)]
            s = (u[0] + u[1]) + (u[2] + u[3])
            mean = _lane_sum(s, bfly) * inv_cd
            d = [uk - mean for uk in u]
            sq = (d[0] * d[0] + d[1] * d[1]) + (d[2] * d[2] + d[3] * d[3])
            var = _lane_sum(sq, bfly) * inv_cd
            a = _rsqrt_vec(var + jnp.float32(1e-5))
            for k in range(4):
                buf_v[r, pl.ds(t * CD + L * k, L)] = d[k] * (a * w_regs[k]) + b_regs[k]
        return carry

    for c in range(NCHUNKS):
        rowbase = base + c * CHUNK
        pltpu.sync_copy(ids_hbm.at[pl.ds(rowbase, CHUNK)], idx_v)
        pltpu.async_copy(embed_hbm.at[idx_v], buf_v, sem).wait()
        lax.fori_loop(0, CHUNK, row_body, 0)
        pltpu.sync_copy(buf_v, out_hbm.at[pl.ds(rowbase, CHUNK)])


_sc_call = functools.partial(
    pl.kernel,
    out_type=jax.ShapeDtypeStruct((B, D), jnp.float32),
    mesh=plsc.VectorSubcoreMesh(core_axis_name="c", subcore_axis_name="s"),
    scratch_types=[
        pltpu.VMEM((CHUNK,), jnp.int32),
        pltpu.VMEM((CHUNK, D), jnp.float32),
        pltpu.VMEM((2 * CD,), jnp.float32),
        pltpu.SemaphoreType.DMA,
    ],
)(_sc_body)


def kernel(class_ids, embed, ln_weight, ln_bias):
    ids = class_ids.astype(jnp.int32)
    out = _sc_call(ids, ln_weight, ln_bias, embed)
    return out.reshape(B, TOK, CD)
